# Initial kernel scaffold; baseline (speedup 1.0000x reference)
import jax
import jax.numpy as jnp
from jax.experimental import pallas as pl


def kernel(node_features, hedge_features, node_senders, node_receivers, node_convolution, hedge2node_senders, hedge2node_receivers, hedge2node_convolution, W_msg, b_msg, W_scale, b_scale):
    raise NotImplementedError("write your pallas kernel here")


# speedup vs baseline: 4.3720x; 4.3720x over previous
import functools

import jax
import jax.numpy as jnp
from jax import lax
from jax.experimental import pallas as pl
from jax.experimental.pallas import tpu as pltpu
from jax.experimental.pallas import tpu_sc as plsc

NC = 2
NS = 16
LANES = 16

K = 128


def _sc_accumulate(n_nodes, n_hedge_feat, e_node, e_hedge,
                   node_features, hedge_features,
                   node_senders, node_receivers, node_conv,
                   h2n_senders, h2n_receivers, h2n_conv):
  N = n_nodes
  DIN = node_features.shape[1]
  DH = n_hedge_feat
  nblk_node = e_node // K
  nblk_hedge = e_hedge // K
  G = N // 8
  ZC = 13

  mesh = plsc.VectorSubcoreMesh(core_axis_name="c", subcore_axis_name="s",
                                num_cores=NC, num_subcores=NS)

  @functools.partial(
      pl.kernel,
      out_type=[
          jax.ShapeDtypeStruct((NC * N, DIN), jnp.float32),
          jax.ShapeDtypeStruct((NC * N, 16), jnp.float32),
          jax.ShapeDtypeStruct((NC * N, DH), jnp.float32),
      ],
      mesh=mesh,
      scratch_types=[
          pltpu.VMEM((K,), jnp.int32),
          pltpu.VMEM((K,), jnp.int32),
          pltpu.VMEM((K,), jnp.float32),
          pltpu.VMEM((K, DIN), jnp.float32),
          pltpu.VMEM((K, 16), jnp.float32),
          pltpu.VMEM((K, DH), jnp.float32),
          pltpu.VMEM_SHARED((N, DIN), jnp.float32),
          pltpu.VMEM_SHARED((N, 16), jnp.float32),
          pltpu.VMEM_SHARED((N, DH), jnp.float32),
          pltpu.SemaphoreType.DMA,
      ],
      compiler_params=pltpu.CompilerParams(use_tc_tiling_on_sc=False),
  )
  def sc_kernel(nf_hbm, hf_hbm, snd_hbm, rcv_hbm, cnv_hbm,
                snd2_hbm, rcv2_hbm, cnv2_hbm,
                p_msg, p_cnt, p_hdg,
                snd_v, rcv_v, conv_v, rows_v, cnt_v, rows2_v,
                acc, accc, acc2, sem):
    c = lax.axis_index("c")
    s = lax.axis_index("s")
    zero16 = jnp.zeros((LANES,), jnp.float32)
    lane_iota = lax.broadcasted_iota(jnp.int32, (LANES,), 0)
    lane0 = lane_iota == 0
    lane1 = lane_iota == 1

    def splat(vec, l):
      return vec.at[jnp.full((LANES,), l, jnp.int32)].get(
          mode="promise_in_bounds")

    def zrows_body(i, _):
      r = i // (DIN // LANES)
      j = i % (DIN // LANES)
      rows_v[r, pl.ds(j * LANES, LANES)] = zero16
      return 0
    lax.fori_loop(0, K * (DIN // LANES), zrows_body, 0)

    def zcnt_body(i, _):
      cnt_v[i, :] = zero16
      return 0
    lax.fori_loop(0, K, zcnt_body, 0)

    g_lo = (s * G) // NS
    g_hi = ((s + 1) * G) // NS
    nchunk = (g_hi - g_lo) // ZC

    def zchunk_body(k, _):
      r = (g_lo + k * ZC) * 8
      pltpu.sync_copy(rows_v.at[pl.ds(0, ZC * 8)], acc.at[pl.ds(r, ZC * 8)])
      pltpu.sync_copy(cnt_v.at[pl.ds(0, ZC * 8)], accc.at[pl.ds(r, ZC * 8)])
      pltpu.sync_copy(cnt_v.at[pl.ds(0, ZC * 8)], acc2.at[pl.ds(r, ZC * 8)])
      return 0
    lax.fori_loop(0, nchunk, zchunk_body, 0)

    def zrem_body(g, _):
      r = g * 8
      pltpu.sync_copy(rows_v.at[pl.ds(0, 8)], acc.at[pl.ds(r, 8)])
      pltpu.sync_copy(cnt_v.at[pl.ds(0, 8)], accc.at[pl.ds(r, 8)])
      pltpu.sync_copy(cnt_v.at[pl.ds(0, 8)], acc2.at[pl.ds(r, 8)])
      return 0
    lax.fori_loop(g_lo + nchunk * ZC, g_hi, zrem_body, 0)
    plsc.subcore_barrier()

    blk_per_sc = nblk_node // NC
    b_lo = c * blk_per_sc + (s * blk_per_sc) // NS
    b_hi = c * blk_per_sc + ((s + 1) * blk_per_sc) // NS

    def node_block(b, _):
      off = b * K
      pltpu.sync_copy(snd_hbm.at[pl.ds(off, K)], snd_v)
      pltpu.sync_copy(rcv_hbm.at[pl.ds(off, K)], rcv_v)
      pltpu.sync_copy(cnv_hbm.at[pl.ds(off, K)], conv_v)
      pltpu.async_copy(nf_hbm.at[snd_v], rows_v, sem).wait()

      def scale_group(kk, _):
        e0 = kk * LANES
        cvec = conv_v[pl.ds(e0, LANES)]
        for l in range(LANES):
          cs = splat(cvec, l)
          e = e0 + l
          cnt_v[e, :] = jnp.where(lane0, cs, 0.0)
          for j in range(DIN // LANES):
            rows_v[e, pl.ds(j * LANES, LANES)] = (
                rows_v[e, pl.ds(j * LANES, LANES)] * cs)
        return 0
      lax.fori_loop(0, K // LANES, scale_group, 0)

      pltpu.sync_copy(rows_v, acc.at[rcv_v], add=True)
      pltpu.sync_copy(cnt_v, accc.at[rcv_v], add=True)
      return 0
    lax.fori_loop(b_lo, b_hi, node_block, 0)

    blk2_per_sc = nblk_hedge // NC
    b2_lo = c * blk2_per_sc + (s * blk2_per_sc) // NS
    b2_hi = c * blk2_per_sc + ((s + 1) * blk2_per_sc) // NS

    def hedge_block(b, _):
      off = b * K
      pltpu.sync_copy(snd2_hbm.at[pl.ds(off, K)], snd_v)
      pltpu.sync_copy(rcv2_hbm.at[pl.ds(off, K)], rcv_v)
      pltpu.sync_copy(cnv2_hbm.at[pl.ds(off, K)], conv_v)
      pltpu.async_copy(hf_hbm.at[snd_v], rows2_v, sem).wait()

      def scale_group(kk, _):
        e0 = kk * LANES
        cvec = conv_v[pl.ds(e0, LANES)]
        for l in range(LANES):
          cs = splat(cvec, l)
          e = e0 + l
          cnt_v[e, :] = jnp.where(lane1, cs, 0.0)
          rows2_v[e, :] = rows2_v[e, :] * cs
        return 0
      lax.fori_loop(0, K // LANES, scale_group, 0)

      pltpu.sync_copy(rows2_v, acc2.at[rcv_v], add=True)
      pltpu.sync_copy(cnt_v, accc.at[rcv_v], add=True)
      return 0
    lax.fori_loop(b2_lo, b2_hi, hedge_block, 0)

    plsc.subcore_barrier()

    def ochunk_body(k, _):
      r = (g_lo + k * ZC) * 8
      o = c * N + r
      pltpu.sync_copy(acc.at[pl.ds(r, ZC * 8)], p_msg.at[pl.ds(o, ZC * 8)])
      pltpu.sync_copy(accc.at[pl.ds(r, ZC * 8)], p_cnt.at[pl.ds(o, ZC * 8)])
      pltpu.sync_copy(acc2.at[pl.ds(r, ZC * 8)], p_hdg.at[pl.ds(o, ZC * 8)])
      return 0
    lax.fori_loop(0, nchunk, ochunk_body, 0)

    def orem_body(g, _):
      r = g * 8
      o = c * N + r
      pltpu.sync_copy(acc.at[pl.ds(r, 8)], p_msg.at[pl.ds(o, 8)])
      pltpu.sync_copy(accc.at[pl.ds(r, 8)], p_cnt.at[pl.ds(o, 8)])
      pltpu.sync_copy(acc2.at[pl.ds(r, 8)], p_hdg.at[pl.ds(o, 8)])
      return 0
    lax.fori_loop(g_lo + nchunk * ZC, g_hi, orem_body, 0)

  return sc_kernel(node_features, hedge_features,
                   node_senders, node_receivers, node_conv,
                   h2n_senders, h2n_receivers, h2n_conv)


def _tc_combine(p_msg, p_cnt, p_hdg, W_msg, b_msg, W_scale, b_scale):
  N2, DIN = p_msg.shape
  N = N2 // NC
  DH = p_hdg.shape[1]
  DOUT = W_msg.shape[0]
  R = 1000
  nblk = N // R

  def body(m0, m1, c0, c1, h0, h1, wm, bm, ws, bs, out):
    a = m0[...] + m1[...]
    cnt = c0[...][:, 0:2] + c1[...][:, 0:2]
    hdg = h0[...] + h1[...]
    msg = lax.dot_general(a, wm[...], (((1,), (1,)), ((), ())),
                          precision=lax.Precision.HIGHEST,
                          preferred_element_type=jnp.float32)
    msg = msg + cnt[:, 0:1] * bm[...]
    scl = lax.dot_general(hdg, ws[...], (((1,), (1,)), ((), ())),
                          precision=lax.Precision.HIGHEST,
                          preferred_element_type=jnp.float32)
    scl = scl + cnt[:, 1:2] * bs[...]
    x = scl * msg
    ax = jnp.abs(x)
    t = jnp.exp(-2.0 * ax)
    out[...] = jnp.sign(x) * (1.0 - t) / (1.0 + t)

  return pl.pallas_call(
      body,
      grid=(nblk,),
      in_specs=[
          pl.BlockSpec((R, DIN), lambda i: (i, 0)),
          pl.BlockSpec((R, DIN), lambda i: (i + nblk, 0)),
          pl.BlockSpec((R, 16), lambda i: (i, 0)),
          pl.BlockSpec((R, 16), lambda i: (i + nblk, 0)),
          pl.BlockSpec((R, DH), lambda i: (i, 0)),
          pl.BlockSpec((R, DH), lambda i: (i + nblk, 0)),
          pl.BlockSpec((DOUT, DIN), lambda i: (0, 0)),
          pl.BlockSpec((1, DOUT), lambda i: (0, 0)),
          pl.BlockSpec((DOUT, DH), lambda i: (0, 0)),
          pl.BlockSpec((1, DOUT), lambda i: (0, 0)),
      ],
      out_specs=pl.BlockSpec((R, DOUT), lambda i: (i, 0)),
      out_shape=jax.ShapeDtypeStruct((N, DOUT), jnp.float32),
  )(p_msg, p_msg, p_cnt, p_cnt, p_hdg, p_hdg,
    W_msg, b_msg.reshape(1, DOUT), W_scale, b_scale.reshape(1, DOUT))


@jax.jit
def kernel(node_features, hedge_features, node_senders, node_receivers,
           node_convolution, hedge2node_senders, hedge2node_receivers,
           hedge2node_convolution, W_msg, b_msg, W_scale, b_scale):
  N, DIN = node_features.shape
  DH = hedge_features.shape[1]
  E = node_senders.shape[0]
  E2 = hedge2node_senders.shape[0]

  p_msg, p_cnt, p_hdg = _sc_accumulate(
      N, DH, E, E2,
      node_features, hedge_features,
      node_senders, node_receivers, node_convolution.reshape(E),
      hedge2node_senders, hedge2node_receivers,
      hedge2node_convolution.reshape(E2))

  return _tc_combine(p_msg, p_cnt, p_hdg, W_msg, b_msg, W_scale, b_scale)


# speedup vs baseline: 6.8381x; 1.5641x over previous
import functools

import jax
import jax.numpy as jnp
from jax import lax
from jax.experimental import pallas as pl
from jax.experimental.pallas import tpu as pltpu
from jax.experimental.pallas import tpu_sc as plsc

NC = 2
NS = 16
LANES = 16

K = 80


def _sc_accumulate(n_nodes, n_hedge_feat, e_node, e_hedge,
                   node_features, hedge_features,
                   node_senders, node_receivers, node_conv,
                   h2n_senders, h2n_receivers, h2n_conv):
  N = n_nodes
  DIN = node_features.shape[1]
  DH = n_hedge_feat
  nblk = e_node // K
  bpt = nblk // (NC * NS)
  assert bpt * NC * NS == nblk and nblk * K == e_node
  assert e_hedge == e_node
  npair = bpt // 2
  G = N // 8
  ZC = 10

  mesh = plsc.VectorSubcoreMesh(core_axis_name="c", subcore_axis_name="s",
                                num_cores=NC, num_subcores=NS)

  @functools.partial(
      pl.kernel,
      out_type=[
          jax.ShapeDtypeStruct((NC * N, DIN), jnp.float32),
          jax.ShapeDtypeStruct((NC * N, 16), jnp.float32),
          jax.ShapeDtypeStruct((NC * N, DH), jnp.float32),
      ],
      mesh=mesh,
      scratch_types=[
          pltpu.VMEM((K,), jnp.int32),
          pltpu.VMEM((K,), jnp.int32),
          pltpu.VMEM((K,), jnp.int32),
          pltpu.VMEM((K,), jnp.int32),
          pltpu.VMEM((K,), jnp.float32),
          pltpu.VMEM((K,), jnp.float32),
          pltpu.VMEM((K, DIN), jnp.float32),
          pltpu.VMEM((K, DIN), jnp.float32),
          pltpu.VMEM((K, 16), jnp.float32),
          pltpu.VMEM((K, 16), jnp.float32),
          pltpu.VMEM((K, DH), jnp.float32),
          pltpu.VMEM((K, DH), jnp.float32),
          pltpu.VMEM_SHARED((N, DIN), jnp.float32),
          pltpu.VMEM_SHARED((N, 16), jnp.float32),
          pltpu.VMEM_SHARED((N, DH), jnp.float32),
          pltpu.SemaphoreType.DMA,
          pltpu.SemaphoreType.DMA,
          pltpu.SemaphoreType.DMA,
          pltpu.SemaphoreType.DMA,
          pltpu.SemaphoreType.DMA,
          pltpu.SemaphoreType.DMA,
      ],
      compiler_params=pltpu.CompilerParams(use_tc_tiling_on_sc=False),
  )
  def sc_kernel(nf_hbm, hf_hbm, snd_hbm, rcv_hbm, cnv_hbm,
                snd2_hbm, rcv2_hbm, cnv2_hbm,
                p_msg, p_cnt, p_hdg,
                snd0, snd1, rcv0, rcv1, cnv0, cnv1,
                rows0, rows1, cnt0, cnt1, hrows0, hrows1,
                acc, accc, acc2,
                semi0, semi1, semg0, semg1, semw0, semw1):
    c = lax.axis_index("c")
    s = lax.axis_index("s")
    zero16 = jnp.zeros((LANES,), jnp.float32)
    lane_iota = lax.broadcasted_iota(jnp.int32, (LANES,), 0)
    lane0 = lane_iota == 0
    lane1 = lane_iota == 1

    def splat(vec, l):
      return vec.at[jnp.full((LANES,), l, jnp.int32)].get(
          mode="promise_in_bounds")

    def zrows_body(i, _):
      r = i // (DIN // LANES)
      j = i % (DIN // LANES)
      rows0[r, pl.ds(j * LANES, LANES)] = zero16
      return 0
    lax.fori_loop(0, K * (DIN // LANES), zrows_body, 0)

    def zcnt_body(i, _):
      cnt0[i, :] = zero16
      return 0
    lax.fori_loop(0, K, zcnt_body, 0)

    g_lo = (s * G) // NS
    g_hi = ((s + 1) * G) // NS
    nchunk = (g_hi - g_lo) // ZC

    def zchunk_body(k, _):
      r = (g_lo + k * ZC) * 8
      pltpu.sync_copy(rows0, acc.at[pl.ds(r, ZC * 8)])
      pltpu.sync_copy(cnt0, accc.at[pl.ds(r, ZC * 8)])
      pltpu.sync_copy(cnt0, acc2.at[pl.ds(r, ZC * 8)])
      return 0
    lax.fori_loop(0, nchunk, zchunk_body, 0)

    def zrem_body(g, _):
      r = g * 8
      pltpu.sync_copy(rows0.at[pl.ds(0, 8)], acc.at[pl.ds(r, 8)])
      pltpu.sync_copy(cnt0.at[pl.ds(0, 8)], accc.at[pl.ds(r, 8)])
      pltpu.sync_copy(cnt0.at[pl.ds(0, 8)], acc2.at[pl.ds(r, 8)])
      return 0
    lax.fori_loop(g_lo + nchunk * ZC, g_hi, zrem_body, 0)
    plsc.subcore_barrier()

    b_base = (c * NS + s) * bpt

    def make_phase(feat, se_hbm, re_hbm, cv_hbm, rbufs, mask, accd):
      dw = rbufs[0].shape[1]
      snds, rcvs, cnvs = (snd0, snd1), (rcv0, rcv1), (cnv0, cnv1)
      cnts = (cnt0, cnt1)
      semis, semgs, semws = (semi0, semi1), (semg0, semg1), (semw0, semw1)

      def fetch(i, u):
        off = (b_base + i) * K
        a = pltpu.async_copy(se_hbm.at[pl.ds(off, K)], snds[u], semis[u])
        b = pltpu.async_copy(re_hbm.at[pl.ds(off, K)], rcvs[u], semis[u])
        d = pltpu.async_copy(cv_hbm.at[pl.ds(off, K)], cnvs[u], semis[u])
        a.wait(); b.wait(); d.wait()
        pltpu.async_copy(feat.at[snds[u]], rbufs[u], semgs[u])

      def wait_g(u):
        pltpu.make_async_copy(feat.at[snds[u]], rbufs[u], semgs[u]).wait()

      def scale(u):
        rows = rbufs[u]
        cnt = cnts[u]
        cnv = cnvs[u]

        def grp(kk, _):
          cvec = cnv[pl.ds(kk * LANES, LANES)]
          for l in range(LANES):
            cs = splat(cvec, l)
            e = kk * LANES + l
            cnt[e, :] = jnp.where(mask, cs, 0.0)
            for j in range(dw // LANES):
              rows[e, pl.ds(j * LANES, LANES)] = (
                  rows[e, pl.ds(j * LANES, LANES)] * cs)
          return 0
        lax.fori_loop(0, K // LANES, grp, 0)

      def scatter(u):
        pltpu.async_copy(rbufs[u], accd.at[rcvs[u]], semws[u], add=True)
        pltpu.async_copy(cnts[u], accc.at[rcvs[u]], semws[u], add=True)

      def wait_w(u):
        pltpu.make_async_copy(rbufs[u], accd.at[rcvs[u]], semws[u]).wait()
        pltpu.make_async_copy(cnts[u], accc.at[rcvs[u]], semws[u]).wait()

      fetch(0, 0)
      fetch(1, 1)

      def pair(t, _):
        wait_g(0)
        scale(0)
        scatter(0)
        wait_g(1)
        scale(1)
        scatter(1)

        @pl.when(t < npair - 1)
        def _():
          wait_w(0)
          fetch(2 * t + 2, 0)
          wait_w(1)
          fetch(2 * t + 3, 1)

        @pl.when(t == npair - 1)
        def _():
          wait_w(0)
          fetch(bpt - 1, 0)
          wait_w(1)
        return 0

      lax.fori_loop(0, npair, pair, 0)

      wait_g(0)
      scale(0)
      scatter(0)
      wait_w(0)

    make_phase(nf_hbm, snd_hbm, rcv_hbm, cnv_hbm, (rows0, rows1),
               lane0, acc)
    make_phase(hf_hbm, snd2_hbm, rcv2_hbm, cnv2_hbm, (hrows0, hrows1),
               lane1, acc2)

    plsc.subcore_barrier()

    def ochunk_body(k, _):
      r = (g_lo + k * ZC) * 8
      o = c * N + r
      pltpu.sync_copy(acc.at[pl.ds(r, ZC * 8)], p_msg.at[pl.ds(o, ZC * 8)])
      pltpu.sync_copy(accc.at[pl.ds(r, ZC * 8)], p_cnt.at[pl.ds(o, ZC * 8)])
      pltpu.sync_copy(acc2.at[pl.ds(r, ZC * 8)], p_hdg.at[pl.ds(o, ZC * 8)])
      return 0
    lax.fori_loop(0, nchunk, ochunk_body, 0)

    def orem_body(g, _):
      r = g * 8
      o = c * N + r
      pltpu.sync_copy(acc.at[pl.ds(r, 8)], p_msg.at[pl.ds(o, 8)])
      pltpu.sync_copy(accc.at[pl.ds(r, 8)], p_cnt.at[pl.ds(o, 8)])
      pltpu.sync_copy(acc2.at[pl.ds(r, 8)], p_hdg.at[pl.ds(o, 8)])
      return 0
    lax.fori_loop(g_lo + nchunk * ZC, g_hi, orem_body, 0)

  return sc_kernel(node_features, hedge_features,
                   node_senders, node_receivers, node_conv,
                   h2n_senders, h2n_receivers, h2n_conv)


def _tc_combine(p_msg, p_cnt, p_hdg, W_msg, b_msg, W_scale, b_scale):
  N2, DIN = p_msg.shape
  N = N2 // NC
  DH = p_hdg.shape[1]
  DOUT = W_msg.shape[0]
  R = 1000
  nblk = N // R

  def body(m0, m1, c0, c1, h0, h1, wm, bm, ws, bs, out):
    a = m0[...] + m1[...]
    cnt = c0[...][:, 0:2] + c1[...][:, 0:2]
    hdg = h0[...] + h1[...]
    msg = lax.dot_general(a, wm[...], (((1,), (1,)), ((), ())),
                          precision=lax.Precision.HIGHEST,
                          preferred_element_type=jnp.float32)
    msg = msg + cnt[:, 0:1] * bm[...]
    scl = lax.dot_general(hdg, ws[...], (((1,), (1,)), ((), ())),
                          precision=lax.Precision.HIGHEST,
                          preferred_element_type=jnp.float32)
    scl = scl + cnt[:, 1:2] * bs[...]
    out[...] = jnp.tanh(scl * msg)

  return pl.pallas_call(
      body,
      grid=(nblk,),
      in_specs=[
          pl.BlockSpec((R, DIN), lambda i: (i, 0)),
          pl.BlockSpec((R, DIN), lambda i: (i + nblk, 0)),
          pl.BlockSpec((R, 16), lambda i: (i, 0)),
          pl.BlockSpec((R, 16), lambda i: (i + nblk, 0)),
          pl.BlockSpec((R, DH), lambda i: (i, 0)),
          pl.BlockSpec((R, DH), lambda i: (i + nblk, 0)),
          pl.BlockSpec((DOUT, DIN), lambda i: (0, 0)),
          pl.BlockSpec((1, DOUT), lambda i: (0, 0)),
          pl.BlockSpec((DOUT, DH), lambda i: (0, 0)),
          pl.BlockSpec((1, DOUT), lambda i: (0, 0)),
      ],
      out_specs=pl.BlockSpec((R, DOUT), lambda i: (i, 0)),
      out_shape=jax.ShapeDtypeStruct((N, DOUT), jnp.float32),
  )(p_msg, p_msg, p_cnt, p_cnt, p_hdg, p_hdg,
    W_msg, b_msg.reshape(1, DOUT), W_scale, b_scale.reshape(1, DOUT))


@jax.jit
def kernel(node_features, hedge_features, node_senders, node_receivers,
           node_convolution, hedge2node_senders, hedge2node_receivers,
           hedge2node_convolution, W_msg, b_msg, W_scale, b_scale):
  N, DIN = node_features.shape
  DH = hedge_features.shape[1]
  E = node_senders.shape[0]
  E2 = hedge2node_senders.shape[0]

  p_msg, p_cnt, p_hdg = _sc_accumulate(
      N, DH, E, E2,
      node_features, hedge_features,
      node_senders, node_receivers, node_convolution.reshape(E),
      hedge2node_senders, hedge2node_receivers,
      hedge2node_convolution.reshape(E2))

  return _tc_combine(p_msg, p_cnt, p_hdg, W_msg, b_msg, W_scale, b_scale)
